# Initial kernel scaffold; baseline (speedup 1.0000x reference)
import jax
import jax.numpy as jnp
from jax.experimental import pallas as pl


def kernel(source, embedding, pe, W, b, scale):
    raise NotImplementedError("write your pallas kernel here")


# speedup vs baseline: 1.3060x; 1.3060x over previous
import functools

import jax
import jax.numpy as jnp
from jax import lax
from jax.experimental import pallas as pl
from jax.experimental.pallas import tpu as pltpu
from jax.experimental.pallas import tpu_sc as plsc

_NC = 2
_NS = 16
_NW = _NC * _NS

_N_ROWS = 8192
_DIM = 128
_ROWS_PER_W = _N_ROWS // _NW
_CHUNK = 128
_N_CHUNK = _ROWS_PER_W // _CHUNK

@functools.cache
def _sc_gather_kernel():
    mesh = plsc.VectorSubcoreMesh(core_axis_name="c", subcore_axis_name="s")

    @functools.partial(
        pl.kernel,
        out_type=jax.ShapeDtypeStruct((_N_ROWS, _DIM), jnp.float32),
        mesh=mesh,
        scratch_types=[
            pltpu.VMEM((_N_CHUNK, _CHUNK), jnp.int32),
            pltpu.VMEM((_ROWS_PER_W, _DIM), jnp.float32),
            pltpu.SemaphoreType.DMA,
        ],
    )
    def _sc_gather(table_hbm, idx_hbm, out_hbm, idx_v, rows_v, sem):
        wid = lax.axis_index("s") * _NC + lax.axis_index("c")
        pltpu.sync_copy(idx_hbm.at[pl.ds(wid * _N_CHUNK, _N_CHUNK)], idx_v)
        cps = [
            pltpu.async_copy(
                table_hbm.at[idx_v.at[j]],
                rows_v.at[pl.ds(j * _CHUNK, _CHUNK)],
                sem,
            )
            for j in range(_N_CHUNK)
        ]
        for cp in cps:
            cp.wait()
        pltpu.sync_copy(rows_v,
                        out_hbm.at[pl.ds(wid * _ROWS_PER_W, _ROWS_PER_W)])

    return _sc_gather


_BLK = 512
_SEQ_BLK = _BLK // 4


def _tc_body(emb_ref, wt_ref, pe_ref, b_ref, sc_ref, o_ref):
    x = jnp.dot(emb_ref[...], wt_ref[...], preferred_element_type=jnp.float32)
    r_io = lax.broadcasted_iota(jnp.int32, (_BLK, _DIM), 0)
    c_io = lax.broadcasted_iota(jnp.int32, (_BLK, _DIM), 1)
    sel = (r_io // 4 == c_io).astype(jnp.float32)
    x = x + jnp.dot(sel, pe_ref[...], preferred_element_type=jnp.float32)
    x = x + b_ref[...]
    lane0 = c_io == 0
    x0 = jnp.sum(jnp.where(lane0, x, 0.0), axis=1, keepdims=True)
    esc = jnp.exp(jnp.broadcast_to(sc_ref[0, 0], x0.shape))
    t = esc / (1.0 + jnp.exp(-x0)) + 1.1
    denom = jnp.sum(jnp.where(lane0, 0.0, x * x), axis=1, keepdims=True)
    denom = jnp.maximum(denom, 1e-8)
    s = (t * t - 1.0) / denom
    o_ref[...] = jnp.where(lane0, t, x * jnp.sqrt(s))


def _tc_compute(emb, wt, pe2, b2, scl):
    return pl.pallas_call(
        _tc_body,
        grid=(_N_ROWS // _BLK,),
        in_specs=[
            pl.BlockSpec((_BLK, _DIM), lambda i: (i, 0)),
            pl.BlockSpec((_DIM, _DIM), lambda i: (0, 0)),
            pl.BlockSpec((_SEQ_BLK, _DIM), lambda i: (i, 0)),
            pl.BlockSpec((1, _DIM), lambda i: (0, 0)),
            pl.BlockSpec(memory_space=pltpu.SMEM),
        ],
        out_specs=pl.BlockSpec((_BLK, _DIM), lambda i: (i, 0)),
        out_shape=jax.ShapeDtypeStruct((_N_ROWS, _DIM), jnp.float32),
    )(emb, wt, pe2, b2, scl)


def kernel(source, embedding, pe, W, b, scale):
    seq, batch, nfeat = source.shape
    n = seq * batch * nfeat
    idx = source.reshape(n // _CHUNK, _CHUNK).astype(jnp.int32)
    rows = _sc_gather_kernel()(embedding, idx)
    pe2 = pe[:seq].reshape(seq, _DIM)
    out = _tc_compute(rows, W.T, pe2, b.reshape(1, _DIM),
                      scale.reshape(1, 1))
    return out.reshape(seq, batch, _DIM)


# speedup vs baseline: 1.3116x; 1.0042x over previous
import functools

import jax
import jax.numpy as jnp
from jax import lax
from jax.experimental import pallas as pl
from jax.experimental.pallas import tpu as pltpu
from jax.experimental.pallas import tpu_sc as plsc

_NC = 2
_NS = 16
_NW = _NC * _NS

_N_ROWS = 8192
_DIM = 128
_ROWS_PER_W = _N_ROWS // _NW
_CHUNK = 128
_N_CHUNK = _ROWS_PER_W // _CHUNK

@functools.cache
def _sc_gather_kernel():
    mesh = plsc.VectorSubcoreMesh(core_axis_name="c", subcore_axis_name="s")

    @functools.partial(
        pl.kernel,
        out_type=jax.ShapeDtypeStruct((_N_ROWS, _DIM), jnp.float32),
        mesh=mesh,
        scratch_types=[
            pltpu.VMEM((_N_CHUNK, _CHUNK), jnp.int32),
            pltpu.VMEM((_ROWS_PER_W, _DIM), jnp.float32),
            pltpu.SemaphoreType.DMA,
        ],
    )
    def _sc_gather(table_hbm, idx_hbm, out_hbm, idx_v, rows_v, sem):
        wid = lax.axis_index("s") * _NC + lax.axis_index("c")
        pltpu.sync_copy(idx_hbm.at[pl.ds(wid * _N_CHUNK, _N_CHUNK)], idx_v)
        cps = [
            pltpu.async_copy(
                table_hbm.at[idx_v.at[j]],
                rows_v.at[pl.ds(j * _CHUNK, _CHUNK)],
                sem,
            )
            for j in range(_N_CHUNK)
        ]
        for cp in cps:
            cp.wait()
        pltpu.sync_copy(rows_v,
                        out_hbm.at[pl.ds(wid * _ROWS_PER_W, _ROWS_PER_W)])

    return _sc_gather


_BLK = 512
_SEQ_BLK = _BLK // 4


def _tc_body(emb_ref, w_ref, pe_ref, b_ref, sc_ref, o_ref):
    x = lax.dot_general(emb_ref[...], w_ref[...], (((1,), (1,)), ((), ())),
                        preferred_element_type=jnp.float32)
    r_io = lax.broadcasted_iota(jnp.int32, (_BLK, _DIM), 0)
    c_io = lax.broadcasted_iota(jnp.int32, (_BLK, _DIM), 1)
    sel = (r_io // 4 == c_io).astype(jnp.float32)
    x = x + jnp.dot(sel, pe_ref[...], preferred_element_type=jnp.float32)
    x = x + b_ref[...]
    lane0 = c_io == 0
    x0 = jnp.sum(jnp.where(lane0, x, 0.0), axis=1, keepdims=True)
    esc = jnp.exp(jnp.broadcast_to(sc_ref[0, 0], x0.shape))
    t = esc / (1.0 + jnp.exp(-x0)) + 1.1
    denom = jnp.sum(jnp.where(lane0, 0.0, x * x), axis=1, keepdims=True)
    denom = jnp.maximum(denom, 1e-8)
    s = (t * t - 1.0) / denom
    o_ref[...] = jnp.where(lane0, t, x * jnp.sqrt(s))


def _tc_compute(emb, w, pe, b2, scl):
    return pl.pallas_call(
        _tc_body,
        grid=(_N_ROWS // _BLK,),
        in_specs=[
            pl.BlockSpec((_BLK, _DIM), lambda i: (i, 0)),
            pl.BlockSpec((_DIM, _DIM), lambda i: (0, 0)),
            pl.BlockSpec((_SEQ_BLK, _DIM), lambda i: (i, 0)),
            pl.BlockSpec((1, _DIM), lambda i: (0, 0)),
            pl.BlockSpec(memory_space=pltpu.SMEM),
        ],
        out_specs=pl.BlockSpec((_BLK, _DIM), lambda i: (i, 0)),
        out_shape=jax.ShapeDtypeStruct((_N_ROWS, _DIM), jnp.float32),
    )(emb, w, pe, b2, scl)


def kernel(source, embedding, pe, W, b, scale):
    seq, batch, nfeat = source.shape
    n = seq * batch * nfeat
    idx = source.reshape(n // _CHUNK, _CHUNK).astype(jnp.int32)
    rows = _sc_gather_kernel()(embedding, idx)
    out = _tc_compute(rows, W, pe.reshape(pe.shape[0], _DIM),
                      b.reshape(1, _DIM), scale.reshape(1, 1))
    return out.reshape(seq, batch, _DIM)


# speedup vs baseline: 1.4461x; 1.1026x over previous
import functools

import jax
import jax.numpy as jnp
from jax import lax
from jax.experimental import pallas as pl
from jax.experimental.pallas import tpu as pltpu
from jax.experimental.pallas import tpu_sc as plsc

_NC = 2
_NS = 16
_NW = _NC * _NS

_N_ROWS = 8192
_DIM = 128
_ROWS_PER_W = _N_ROWS // _NW
_CHUNK = 128
_N_CHUNK = _ROWS_PER_W // _CHUNK

@functools.cache
def _sc_gather_kernel():
    mesh = plsc.VectorSubcoreMesh(core_axis_name="c", subcore_axis_name="s")

    @functools.partial(
        pl.kernel,
        out_type=jax.ShapeDtypeStruct((_N_ROWS, _DIM), jnp.float32),
        mesh=mesh,
        scratch_types=[
            pltpu.VMEM((_N_CHUNK, _CHUNK), jnp.int32),
            pltpu.VMEM((_ROWS_PER_W, _DIM), jnp.float32),
            pltpu.SemaphoreType.DMA,
        ],
    )
    def _sc_gather(table_hbm, idx_hbm, out_hbm, idx_v, rows_v, sem):
        wid = lax.axis_index("s") * _NC + lax.axis_index("c")
        pltpu.sync_copy(idx_hbm.at[pl.ds(wid * _N_CHUNK, _N_CHUNK)], idx_v)
        cps = [
            pltpu.async_copy(
                table_hbm.at[idx_v.at[j]],
                rows_v.at[pl.ds(j * _CHUNK, _CHUNK)],
                sem,
            )
            for j in range(_N_CHUNK)
        ]
        for cp in cps:
            cp.wait()
        pltpu.sync_copy(rows_v,
                        out_hbm.at[pl.ds(wid * _ROWS_PER_W, _ROWS_PER_W)])

    return _sc_gather


_BLK = 1024
_SEQ_BLK = _BLK // 4


def _tc_body(emb_ref, w_ref, pe_ref, b_ref, sc_ref, o_ref):
    x = lax.dot_general(emb_ref[...], w_ref[...], (((1,), (1,)), ((), ())),
                        preferred_element_type=jnp.float32)
    r_io = lax.broadcasted_iota(jnp.int32, (_BLK, _SEQ_BLK), 0)
    q_io = lax.broadcasted_iota(jnp.int32, (_BLK, _SEQ_BLK), 1)
    c_io = lax.broadcasted_iota(jnp.int32, (_BLK, _DIM), 1)
    sel = (r_io // 4 == q_io).astype(jnp.float32)
    x = x + jnp.dot(sel, pe_ref[...], preferred_element_type=jnp.float32)
    x = x + b_ref[...]
    k_io = lax.broadcasted_iota(jnp.int32, (_DIM, _DIM), 0)
    m_io = lax.broadcasted_iota(jnp.int32, (_DIM, _DIM), 1)
    first = (k_io == 0)
    red_m = jnp.where(m_io == 0, first.astype(jnp.float32), 0.0)
    red_m = jnp.where(m_io == 1, jnp.where(first, 0.0, 1.0), red_m)
    red = jnp.dot(jnp.where(c_io == 0, x, x * x), red_m,
                  preferred_element_type=jnp.float32)
    x0 = red[:, :1]
    denom = jnp.maximum(red[:, 1:2], 1e-8)
    esc = jnp.exp(jnp.broadcast_to(sc_ref[0, 0], x0.shape))
    t = esc / (1.0 + jnp.exp(-x0)) + 1.1
    s = (t * t - 1.0) / denom
    o_ref[...] = jnp.where(c_io == 0, t, x * jnp.sqrt(s))


def _tc_compute(emb, w, pe, b2, scl):
    return pl.pallas_call(
        _tc_body,
        grid=(_N_ROWS // _BLK,),
        in_specs=[
            pl.BlockSpec((_BLK, _DIM), lambda i: (i, 0)),
            pl.BlockSpec((_DIM, _DIM), lambda i: (0, 0)),
            pl.BlockSpec((_SEQ_BLK, _DIM), lambda i: (i, 0)),
            pl.BlockSpec((1, _DIM), lambda i: (0, 0)),
            pl.BlockSpec(memory_space=pltpu.SMEM),
        ],
        out_specs=pl.BlockSpec((_BLK, _DIM), lambda i: (i, 0)),
        out_shape=jax.ShapeDtypeStruct((_N_ROWS, _DIM), jnp.float32),
    )(emb, w, pe, b2, scl)


def kernel(source, embedding, pe, W, b, scale):
    seq, batch, nfeat = source.shape
    n = seq * batch * nfeat
    idx = source.reshape(n // _CHUNK, _CHUNK).astype(jnp.int32)
    rows = _sc_gather_kernel()(embedding, idx)
    out = _tc_compute(rows, W, pe.reshape(pe.shape[0], _DIM),
                      b.reshape(1, _DIM), scale.reshape(1, 1))
    return out.reshape(seq, batch, _DIM)
